# Initial kernel scaffold; baseline (speedup 1.0000x reference)
import jax
import jax.numpy as jnp
from jax.experimental import pallas as pl


def kernel(x, edge_index, edge_attr, W1, b1, W2, b2, gamma, beta):
    raise NotImplementedError("write your pallas kernel here")


# speedup vs baseline: 5.2476x; 5.2476x over previous
import functools

import jax
import jax.numpy as jnp
from jax import lax
from jax.experimental import pallas as pl
from jax.experimental.pallas import tpu as pltpu
from jax.experimental.pallas import tpu_sc as plsc

NC = 2
NS = 16
NW = NC * NS
CH = 64


def _sc_mesh():
    return plsc.VectorSubcoreMesh(
        core_axis_name="c", subcore_axis_name="s",
        num_cores=NC, num_subcores=NS)


def _make_sc_rows(nacc, nchunk, d):
    @functools.partial(
        pl.kernel,
        out_type=jax.ShapeDtypeStruct((NC, nacc, d), jnp.float32),
        mesh=_sc_mesh(),
        scratch_types=[
            pltpu.VMEM_SHARED((nacc, d), jnp.float32),
            pltpu.VMEM((1, CH), jnp.int32),
            pltpu.VMEM((1, CH), jnp.int32),
            pltpu.VMEM((CH, d), jnp.float32),
            pltpu.SemaphoreType.DMA,
        ],
    )
    def kern(rows_hbm, src_hbm, dst_hbm, zs_hbm, sp_hbm,
             s_acc, src_v, dst_v, rows_v, sem):
        c = lax.axis_index("c")
        s = lax.axis_index("s")
        w = s * NC + c

        @pl.when(s == 0)
        def _():
            pltpu.sync_copy(zs_hbm, s_acc)

        plsc.subcore_barrier()

        @pl.loop(0, nchunk)
        def body(j):
            ci = w * nchunk + j
            pltpu.sync_copy(src_hbm.at[ci], src_v)
            pltpu.sync_copy(dst_hbm.at[ci], dst_v)
            pltpu.async_copy(rows_hbm.at[src_v.at[0]], rows_v, sem).wait()
            pltpu.sync_copy(rows_v, s_acc.at[dst_v.at[0]], add=True)

        plsc.subcore_barrier()

        @pl.when(s == 0)
        def _():
            pltpu.sync_copy(s_acc, sp_hbm.at[c])

    return kern


def _make_sc_attr(nacc, nchunk, da2):
    @functools.partial(
        pl.kernel,
        out_type=jax.ShapeDtypeStruct((NC, nacc, da2), jnp.float32),
        mesh=_sc_mesh(),
        scratch_types=[
            pltpu.VMEM_SHARED((nacc, da2), jnp.float32),
            pltpu.VMEM((1, CH), jnp.int32),
            pltpu.VMEM((CH, da2), jnp.float32),
            pltpu.SemaphoreType.DMA,
        ],
    )
    def kern(ea_hbm, dst_hbm, zt_hbm, tp_hbm,
             t_acc, dst_v, ea_v, sem):
        c = lax.axis_index("c")
        s = lax.axis_index("s")
        w = s * NC + c

        @pl.when(s == 0)
        def _():
            pltpu.sync_copy(zt_hbm, t_acc)

        plsc.subcore_barrier()

        @pl.loop(0, nchunk)
        def body(j):
            ci = w * nchunk + j
            pltpu.sync_copy(dst_hbm.at[ci], dst_v)
            pltpu.sync_copy(ea_hbm.at[pl.ds(ci * CH, CH)], ea_v)
            pltpu.sync_copy(ea_v, t_acc.at[dst_v.at[0]], add=True)

        plsc.subcore_barrier()

        @pl.when(s == 0)
        def _():
            pltpu.sync_copy(t_acc, tp_hbm.at[c])

    return kern


def kernel(x, edge_index, edge_attr, W1, b1, W2, b2, gamma, beta):
    n, d = x.shape
    e, de = edge_attr.shape
    h = W1.shape[1]
    f32 = jnp.float32
    da2 = 128

    nchunk = -(-e // (NW * CH))
    e_pad = nchunk * CH * NW
    nacc = -(-(n + 1) // 128) * 128

    src = edge_index[0]
    dst = edge_index[1]
    pad = e_pad - e
    src_p = jnp.concatenate([src, jnp.zeros((pad,), jnp.int32)])
    dst_p = jnp.concatenate([dst, jnp.full((pad,), n, jnp.int32)])
    src_b = src_p.reshape(NW * nchunk, 1, CH)
    dst_b = dst_p.reshape(NW * nchunk, 1, CH)
    ea2 = jnp.concatenate(
        [edge_attr, jnp.ones((e, 1), f32), jnp.zeros((e, da2 - de - 1), f32)],
        axis=1)
    ea2 = jnp.concatenate([ea2, jnp.zeros((pad, da2), f32)], axis=0)

    z_s = jnp.zeros((nacc, d), f32)
    z_t = jnp.zeros((nacc, da2), f32)

    W1a, W1b, W1c = W1[:d], W1[d:2 * d], W1[2 * d:]
    W2a, W2b, W2c = W2[:h], W2[h:2 * h], W2[2 * h:]
    W1cx = jnp.concatenate(
        [W1c, b1[None, :], jnp.zeros((da2 - de - 1, h), f32)], axis=0)
    W2cx = jnp.concatenate(
        [W2c, b2[None, :], jnp.zeros((da2 - de - 1, h), f32)], axis=0)
    c1 = (W1c.sum(0) + b1)[None, :]
    c2 = (W2c.sum(0) + b2)[None, :]
    gamma2 = gamma[None, :]
    beta2 = beta[None, :]

    sp1 = _make_sc_rows(nacc, nchunk, d)(x, src_b, dst_b, z_s)
    tp = _make_sc_attr(nacc, nchunk, da2)(ea2, dst_b, z_t)

    nb = 5
    bn = n // nb
    assert bn * nb == n and bn % 8 == 0

    def tc1a(x_r, sp_r, tp_r, w1a_r, w1b_r, w1cx_r, c1_r,
             h1_r, st_r, acc_sum, acc_sq):
        i = pl.program_id(0)
        xb = x_r[...]
        sb = sp_r[0] + sp_r[1] + xb
        tb = tp_r[0] + tp_r[1]
        deg = tb[:, de:de + 1] + 1.0
        agg = (deg * jnp.dot(xb, w1a_r[...], preferred_element_type=f32)
               + jnp.dot(sb, w1b_r[...], preferred_element_type=f32)
               + jnp.dot(tb, w1cx_r[...], preferred_element_type=f32)
               + c1_r[...])
        h1 = jnp.maximum(agg, 0.0)
        h1_r[...] = h1

        @pl.when(i == 0)
        def _():
            acc_sum[...] = jnp.zeros_like(acc_sum)
            acc_sq[...] = jnp.zeros_like(acc_sq)

        acc_sum[...] += jnp.sum(h1, axis=0, keepdims=True)
        acc_sq[...] += jnp.sum(h1 * h1, axis=0, keepdims=True)

        @pl.when(i == nb - 1)
        def _():
            st_r[0:1, :] = acc_sum[...]
            st_r[1:2, :] = acc_sq[...]

    h1, stats = pl.pallas_call(
        tc1a,
        grid=(nb,),
        in_specs=[
            pl.BlockSpec((bn, d), lambda i: (i, 0)),
            pl.BlockSpec((NC, bn, d), lambda i: (0, i, 0)),
            pl.BlockSpec((NC, bn, da2), lambda i: (0, i, 0)),
            pl.BlockSpec((d, h), lambda i: (0, 0)),
            pl.BlockSpec((d, h), lambda i: (0, 0)),
            pl.BlockSpec((da2, h), lambda i: (0, 0)),
            pl.BlockSpec((1, h), lambda i: (0, 0)),
        ],
        out_specs=[
            pl.BlockSpec((bn, h), lambda i: (i, 0)),
            pl.BlockSpec((8, h), lambda i: (0, 0)),
        ],
        out_shape=[
            jax.ShapeDtypeStruct((n, h), f32),
            jax.ShapeDtypeStruct((8, h), f32),
        ],
        scratch_shapes=[
            pltpu.VMEM((1, h), f32),
            pltpu.VMEM((1, h), f32),
        ],
    )(x, sp1, tp, W1a, W1b, W1cx, c1)

    inv_n = 1.0 / n

    def tc1b(h1_r, st_r, tp_r, g_r, be_r, w2a_r, w2b_r, w2cx_r, c2_r,
             hb_r, p2_r):
        mean = st_r[0:1, :] * inv_n
        var = st_r[1:2, :] * inv_n - mean * mean
        scale = g_r[...] * lax.rsqrt(var + 1e-5)
        shift = be_r[...] - mean * scale
        hh = jnp.maximum(h1_r[...] * scale + shift, 0.0)
        tb = tp_r[0] + tp_r[1]
        deg = tb[:, de:de + 1] + 1.0
        hb = jnp.dot(hh, w2b_r[...], preferred_element_type=f32)
        hb_r[...] = hb
        p2_r[...] = (deg * jnp.dot(hh, w2a_r[...], preferred_element_type=f32)
                     + hb
                     + jnp.dot(tb, w2cx_r[...], preferred_element_type=f32)
                     + c2_r[...])

    hb, p2 = pl.pallas_call(
        tc1b,
        grid=(nb,),
        in_specs=[
            pl.BlockSpec((bn, h), lambda i: (i, 0)),
            pl.BlockSpec((8, h), lambda i: (0, 0)),
            pl.BlockSpec((NC, bn, da2), lambda i: (0, i, 0)),
            pl.BlockSpec((1, h), lambda i: (0, 0)),
            pl.BlockSpec((1, h), lambda i: (0, 0)),
            pl.BlockSpec((h, h), lambda i: (0, 0)),
            pl.BlockSpec((h, h), lambda i: (0, 0)),
            pl.BlockSpec((da2, h), lambda i: (0, 0)),
            pl.BlockSpec((1, h), lambda i: (0, 0)),
        ],
        out_specs=[
            pl.BlockSpec((bn, h), lambda i: (i, 0)),
            pl.BlockSpec((bn, h), lambda i: (i, 0)),
        ],
        out_shape=[
            jax.ShapeDtypeStruct((n, h), f32),
            jax.ShapeDtypeStruct((n, h), f32),
        ],
    )(h1, stats, tp, gamma2, beta2, W2a, W2b, W2cx, c2)

    sp2 = _make_sc_rows(nacc, nchunk, h)(hb, src_b, dst_b, z_s)

    def tc2(sp_r, p2_r, o_r):
        o_r[...] = jnp.maximum(sp_r[0] + sp_r[1] + p2_r[...], 0.0)

    out = pl.pallas_call(
        tc2,
        grid=(nb,),
        in_specs=[
            pl.BlockSpec((NC, bn, h), lambda i: (0, i, 0)),
            pl.BlockSpec((bn, h), lambda i: (i, 0)),
        ],
        out_specs=pl.BlockSpec((bn, h), lambda i: (i, 0)),
        out_shape=jax.ShapeDtypeStruct((n, h), f32),
    )(sp2, p2)
    return out
